# baseline (speedup 1.0000x reference)
import functools

import jax
import jax.numpy as jnp
from jax import lax
from jax.experimental import pallas as pl
from jax.experimental.pallas import tpu as pltpu
from jax.experimental.pallas import tpu_sc as plsc


def kernel(x, scalar, vector_i, vector_j, vector_k):
    B, L = x.shape
    V, D = scalar.shape
    N = B * L
    T = 4
    CB = 128
    BB = B // 128

    info = plsc.get_sparse_core_info()
    NW = info.num_cores * info.num_subcores
    assert N % NW == 0 and B % CB == 0
    n_per_w = N // NW
    units_per_w = n_per_w // CB
    assert units_per_w % 2 == 0
    cb = B // CB

    stacked = jnp.stack((scalar, vector_i, vector_j, vector_k))

    x_lt = x.T.reshape(N).astype(jnp.int32)

    mesh = plsc.VectorSubcoreMesh(core_axis_name="c", subcore_axis_name="s")

    @functools.partial(
        pl.kernel,
        mesh=mesh,
        compiler_params=pltpu.CompilerParams(
            needs_layout_passes=False, use_tc_tiling_on_sc=False),
        out_type=jax.ShapeDtypeStruct((L, D, BB, T, 128), jnp.float32),
        scratch_types=[
            pltpu.VMEM((n_per_w,), jnp.int32),
            pltpu.VMEM((2, T, CB, D), jnp.float32),
            pltpu.VMEM((2, D, 1, T, 128), jnp.float32),
            pltpu.SemaphoreType.DMA,
            pltpu.SemaphoreType.DMA,
            pltpu.SemaphoreType.DMA,
            pltpu.SemaphoreType.DMA,
        ],
    )
    def sc_kernel(x_hbm, tabs_hbm, out_hbm,
                  idx_v, rows2, obuf2, sg0, sg1, so0, so1):
        wid = lax.axis_index("s") * info.num_cores + lax.axis_index("c")
        base = wid * n_per_w
        semg = (sg0, sg1)
        semo = (so0, so1)

        pltpu.sync_copy(x_hbm.at[pl.ds(base, n_per_w)], idx_v)

        lane = lax.broadcasted_iota(jnp.int32, (16,), 0)
        tabs = tuple(tabs_hbm.at[t] for t in range(T))
        d_vecs = [lane + 16 * h for h in range(D // 16)]
        t_splats = [jnp.full((16,), t, jnp.int32) for t in range(T)]
        zero16 = jnp.full((16,), 0, jnp.int32)

        def issue_gathers(j, s):
            idx_slice = idx_v.at[pl.ds(j * CB, CB)]
            for t in range(T):
                pltpu.async_copy(tabs[t].at[idx_slice], rows2.at[s, t],
                                 semg[s])

        def wait_gathers(s):
            idx_slice = idx_v.at[pl.ds(0, CB)]
            for t in range(T):
                pltpu.make_async_copy(tabs[t].at[idx_slice], rows2.at[s, t],
                                      semg[s]).wait()

        def out_slice(j):
            u = wid * units_per_w + j
            return out_hbm.at[u // cb, :, pl.ds(u % cb, 1), :, :]

        issue_gathers(0, 0)
        issue_gathers(1, 1)

        def pair_body(p, carry):
            for s in range(2):
                j = 2 * p + s
                wait_gathers(s)

                @pl.when(j >= 2)
                def _():
                    pltpu.make_async_copy(obuf2.at[s], out_slice(j),
                                          semo[s]).wait()

                @plsc.parallel_loop(0, CB, unroll=4)
                def _(i):
                    bi = jnp.full((16,), i, jnp.int32)
                    for t in range(T):
                        for h in range(D // 16):
                            vals = rows2[s, t, i, pl.ds(16 * h, 16)]
                            plsc.store_scatter(
                                obuf2.at[s],
                                [d_vecs[h], zero16, t_splats[t], bi], vals)

                pltpu.async_copy(obuf2.at[s], out_slice(j), semo[s])

                @pl.when(j + 2 < units_per_w)
                def _():
                    issue_gathers(j + 2, s)

            return carry

        lax.fori_loop(0, units_per_w // 2, pair_body, 0)

        for s in range(2):
            pltpu.make_async_copy(obuf2.at[s],
                                  out_slice(units_per_w - 2 + s),
                                  semo[s]).wait()

    out5 = sc_kernel(x_lt, stacked)
    return out5.transpose(2, 4, 0, 1, 3).reshape(B, L, D, T)


# speedup vs baseline: 1.1928x; 1.1928x over previous
import functools

import jax
import jax.numpy as jnp
from jax import lax
from jax.experimental import pallas as pl
from jax.experimental.pallas import tpu as pltpu
from jax.experimental.pallas import tpu_sc as plsc


def _make_transpose_kernel(V, D, T):
    GV = 512
    n_main = V // GV // 32
    v_main = n_main * 32 * GV
    v_extra = ((V - v_main) // GV) * GV
    v_tail = V - v_main - v_extra
    assert v_tail == 64

    mesh = plsc.VectorSubcoreMesh(core_axis_name="c", subcore_axis_name="s")
    out_t = jax.ShapeDtypeStruct((V * D,), jnp.float32)

    @functools.partial(
        pl.kernel,
        mesh=mesh,
        compiler_params=pltpu.CompilerParams(
            needs_layout_passes=False, use_tc_tiling_on_sc=True),
        out_type=[out_t] * T,
        scratch_types=[
            pltpu.VMEM((4, D, GV), jnp.float32),
            pltpu.VMEM((2, GV * D), jnp.float32),
            pltpu.VMEM((v_tail * D,), jnp.float32),
            pltpu.SemaphoreType.DMA,
            pltpu.SemaphoreType.DMA,
            pltpu.SemaphoreType.DMA,
            pltpu.SemaphoreType.DMA,
            pltpu.SemaphoreType.DMA,
            pltpu.SemaphoreType.DMA,
        ],
    )
    def transpose_kernel(a_t, b_t, c_t, d_t, ta, tb_, tc_, td_,
                         oa, ob, oc, od,
                         buf2, orow2, ptmp, si0, si1, si2, si3, so0, so1):
        wid = lax.axis_index("s") * 2 + lax.axis_index("c")
        tabs = (a_t, b_t, c_t, d_t)
        outs = (oa, ob, oc, od)
        semi = (si0, si1, si2, si3)
        semo = (so0, so1)

        lane = lax.broadcasted_iota(jnp.int32, (16,), 0)
        d_vecs = [lane + 16 * h for h in range(D // 16)]

        def issue_in(t, col0, s):
            for db in range(D // 8):
                for k in range(GV // 128):
                    pltpu.async_copy(
                        tabs[t].at[pl.ds(db * 8, 8),
                                   pl.ds(col0 + k * 128, 128)],
                        buf2.at[s, pl.ds(db * 8, 8), pl.ds(k * 128, 128)],
                        semi[s])

        def wait_in(s):
            for db in range(D // 8):
                for k in range(GV // 128):
                    pltpu.make_async_copy(
                        tabs[0].at[pl.ds(0, 8), pl.ds(0, 128)],
                        buf2.at[s, pl.ds(db * 8, 8), pl.ds(k * 128, 128)],
                        semi[s]).wait()

        def transpose_group(s, so):
            @plsc.parallel_loop(0, GV, unroll=4)
            def _(vl):
                vs = jnp.full((16,), vl, jnp.int32)
                for h in range(D // 16):
                    vals = plsc.load_gather(buf2.at[s], [d_vecs[h], vs])
                    orow2[so, pl.ds(vl * D + 16 * h, 16)] = vals

        def wait_out(t, s):
            pltpu.make_async_copy(orow2.at[s], outs[t].at[pl.ds(0, GV * D)],
                                  semo[s]).wait()

        for t in range(4):
            issue_in(t, wid * GV, t)

        def round_body(p, carry):
            g = p * 32 + wid
            col0 = g * GV
            for t in range(4):
                so = t & 1
                wait_in(t)
                if t < 2:
                    @pl.when(p >= 1)
                    def _():
                        wait_out(t, so)
                else:
                    wait_out(t - 2, so)
                transpose_group(t, so)
                pltpu.async_copy(orow2.at[so],
                                 outs[t].at[pl.ds(col0 * D, GV * D)],
                                 semo[so])

                @pl.when(p < n_main - 1)
                def _():
                    issue_in(t, (g + 32) * GV, t)
            return carry

        lax.fori_loop(0, n_main, round_body, 0)
        for t in range(2):
            wait_out(t + 2, t)

        @pl.when(wid == 0)
        def _():
            for e in range(v_extra // GV):
                col0 = v_main + e * GV
                for t in range(4):
                    issue_in(t, col0, 0)
                    wait_in(0)
                    transpose_group(0, 0)
                    pltpu.sync_copy(orow2.at[0],
                                    outs[t].at[pl.ds(col0 * D, GV * D)])

        @pl.when(wid == 1)
        def _():
            for t, tail in enumerate((ta, tb_, tc_, td_)):
                pltpu.sync_copy(tail, ptmp)
                pltpu.sync_copy(
                    ptmp, outs[t].at[pl.ds((V - v_tail) * D, v_tail * D)])

    return transpose_kernel


def kernel(x, scalar, vector_i, vector_j, vector_k):
    B, L = x.shape
    V, D = scalar.shape
    N = B * L
    T = 4
    CB = 128
    BB = B // 128

    info = plsc.get_sparse_core_info()
    NW = info.num_cores * info.num_subcores
    assert N % NW == 0 and B % CB == 0
    n_per_w = N // NW
    units_per_w = n_per_w // CB
    assert units_per_w % 2 == 0
    cb = B // CB

    tk = _make_transpose_kernel(V, D, T)
    tails = [jnp.reshape(tb[V - 64:], (-1,))
             for tb in (scalar, vector_i, vector_j, vector_k)]
    flats = tk(scalar.T, vector_i.T, vector_j.T, vector_k.T, *tails)
    tab_a, tab_b, tab_c, tab_d = (f.reshape(V, D) for f in flats)

    x_lt = x.T.reshape(N).astype(jnp.int32)

    mesh = plsc.VectorSubcoreMesh(core_axis_name="c", subcore_axis_name="s")

    @functools.partial(
        pl.kernel,
        mesh=mesh,
        compiler_params=pltpu.CompilerParams(
            needs_layout_passes=False, use_tc_tiling_on_sc=False),
        out_type=jax.ShapeDtypeStruct((L, D, BB, T, 128), jnp.float32),
        scratch_types=[
            pltpu.VMEM((n_per_w,), jnp.int32),
            pltpu.VMEM((2, T, CB, D), jnp.float32),
            pltpu.VMEM((2, D, 1, T, 128), jnp.float32),
            pltpu.SemaphoreType.DMA,
            pltpu.SemaphoreType.DMA,
            pltpu.SemaphoreType.DMA,
            pltpu.SemaphoreType.DMA,
        ],
    )
    def sc_kernel(x_hbm, a_hbm, b_hbm, c_hbm, d_hbm, out_hbm,
                  idx_v, rows2, obuf2, sg0, sg1, so0, so1):
        wid = lax.axis_index("s") * info.num_cores + lax.axis_index("c")
        base = wid * n_per_w
        semg = (sg0, sg1)
        semo = (so0, so1)

        pltpu.sync_copy(x_hbm.at[pl.ds(base, n_per_w)], idx_v)

        lane = lax.broadcasted_iota(jnp.int32, (16,), 0)
        tabs = (a_hbm, b_hbm, c_hbm, d_hbm)
        d_vecs = [lane + 16 * h for h in range(D // 16)]
        t_splats = [jnp.full((16,), t, jnp.int32) for t in range(T)]
        zero16 = jnp.full((16,), 0, jnp.int32)

        def issue_gathers(j, s):
            idx_slice = idx_v.at[pl.ds(j * CB, CB)]
            for t in range(T):
                pltpu.async_copy(tabs[t].at[idx_slice], rows2.at[s, t],
                                 semg[s])

        def wait_gathers(s):
            idx_slice = idx_v.at[pl.ds(0, CB)]
            for t in range(T):
                pltpu.make_async_copy(tabs[t].at[idx_slice], rows2.at[s, t],
                                      semg[s]).wait()

        def out_slice(j):
            u = wid * units_per_w + j
            return out_hbm.at[u // cb, :, pl.ds(u % cb, 1), :, :]

        issue_gathers(0, 0)
        issue_gathers(1, 1)

        def pair_body(p, carry):
            for s in range(2):
                j = 2 * p + s
                wait_gathers(s)

                @pl.when(j >= 2)
                def _():
                    pltpu.make_async_copy(obuf2.at[s], out_slice(j),
                                          semo[s]).wait()

                @plsc.parallel_loop(0, CB, unroll=4)
                def _(i):
                    bi = jnp.full((16,), i, jnp.int32)
                    for t in range(T):
                        for h in range(D // 16):
                            vals = rows2[s, t, i, pl.ds(16 * h, 16)]
                            plsc.store_scatter(
                                obuf2.at[s],
                                [d_vecs[h], zero16, t_splats[t], bi], vals)

                pltpu.async_copy(obuf2.at[s], out_slice(j), semo[s])

                @pl.when(j + 2 < units_per_w)
                def _():
                    issue_gathers(j + 2, s)

            return carry

        lax.fori_loop(0, units_per_w // 2, pair_body, 0)

        for s in range(2):
            pltpu.make_async_copy(obuf2.at[s],
                                  out_slice(units_per_w - 2 + s),
                                  semo[s]).wait()

    out5 = sc_kernel(x_lt, tab_a, tab_b, tab_c, tab_d)
    return out5.transpose(2, 4, 0, 1, 3).reshape(B, L, D, T)


# speedup vs baseline: 1.3093x; 1.0977x over previous
import functools

import jax
import jax.numpy as jnp
from jax import lax
from jax.experimental import pallas as pl
from jax.experimental.pallas import tpu as pltpu
from jax.experimental.pallas import tpu_sc as plsc


def kernel(x, scalar, vector_i, vector_j, vector_k):
    B, L = x.shape
    V, D = scalar.shape
    N = B * L
    T = 4
    CB = 128
    BB = B // 128

    info = plsc.get_sparse_core_info()
    NW = info.num_cores * info.num_subcores
    assert N % NW == 0 and B % CB == 0
    n_per_w = N // NW
    units_per_w = n_per_w // CB
    assert units_per_w % 2 == 0
    cb = B // CB

    x_lt = x.T.reshape(N).astype(jnp.int32)

    mesh = plsc.VectorSubcoreMesh(core_axis_name="c", subcore_axis_name="s")

    @functools.partial(
        pl.kernel,
        mesh=mesh,
        compiler_params=pltpu.CompilerParams(
            needs_layout_passes=False, use_tc_tiling_on_sc=False),
        out_type=jax.ShapeDtypeStruct((L, D, BB, T, 128), jnp.float32),
        scratch_types=[
            pltpu.VMEM((n_per_w,), jnp.int32),
            pltpu.VMEM((2, T, CB, D), jnp.float32),
            pltpu.VMEM((2, D, 1, T, 128), jnp.float32),
            pltpu.SemaphoreType.DMA,
            pltpu.SemaphoreType.DMA,
            pltpu.SemaphoreType.DMA,
            pltpu.SemaphoreType.DMA,
        ],
    )
    def sc_kernel(x_hbm, a_hbm, b_hbm, c_hbm, d_hbm, out_hbm,
                  idx_v, rows2, obuf2, sg0, sg1, so0, so1):
        wid = lax.axis_index("s") * info.num_cores + lax.axis_index("c")
        base = wid * n_per_w
        semg = (sg0, sg1)
        semo = (so0, so1)

        pltpu.sync_copy(x_hbm.at[pl.ds(base, n_per_w)], idx_v)

        lane = lax.broadcasted_iota(jnp.int32, (16,), 0)
        tabs = (a_hbm, b_hbm, c_hbm, d_hbm)
        d_vecs = [lane + 16 * h for h in range(D // 16)]
        t_splats = [jnp.full((16,), t, jnp.int32) for t in range(T)]
        zero16 = jnp.full((16,), 0, jnp.int32)

        def issue_gathers(j, s):
            idx_slice = idx_v.at[pl.ds(j * CB, CB)]
            for t in range(T):
                pltpu.async_copy(tabs[t].at[idx_slice], rows2.at[s, t],
                                 semg[s])

        def wait_gathers(s):
            idx_slice = idx_v.at[pl.ds(0, CB)]
            for t in range(T):
                pltpu.make_async_copy(tabs[t].at[idx_slice], rows2.at[s, t],
                                      semg[s]).wait()

        def out_slice(j):
            u = wid * units_per_w + j
            return out_hbm.at[u // cb, :, pl.ds(u % cb, 1), :, :]

        issue_gathers(0, 0)
        issue_gathers(1, 1)

        def pair_body(p, carry):
            for s in range(2):
                j = 2 * p + s
                wait_gathers(s)

                @pl.when(j >= 2)
                def _():
                    pltpu.make_async_copy(obuf2.at[s], out_slice(j),
                                          semo[s]).wait()

                @plsc.parallel_loop(0, CB, unroll=4)
                def _(i):
                    bi = jnp.full((16,), i, jnp.int32)
                    for t in range(T):
                        for h in range(D // 16):
                            vals = rows2[s, t, i, pl.ds(16 * h, 16)]
                            plsc.store_scatter(
                                obuf2.at[s],
                                [d_vecs[h], zero16, t_splats[t], bi], vals)

                pltpu.async_copy(obuf2.at[s], out_slice(j), semo[s])

                @pl.when(j + 2 < units_per_w)
                def _():
                    issue_gathers(j + 2, s)

            return carry

        lax.fori_loop(0, units_per_w // 2, pair_body, 0)

        for s in range(2):
            pltpu.make_async_copy(obuf2.at[s],
                                  out_slice(units_per_w - 2 + s),
                                  semo[s]).wait()

    out5 = sc_kernel(x_lt, scalar, vector_i, vector_j, vector_k)
    return out5.transpose(2, 4, 0, 1, 3).reshape(B, L, D, T)
